# Initial kernel scaffold; baseline (speedup 1.0000x reference)
import jax
import jax.numpy as jnp
from jax.experimental import pallas as pl


def kernel(x, edge_index, W_pred, b_pred):
    raise NotImplementedError("write your pallas kernel here")


# speedup vs baseline: 3.4545x; 3.4545x over previous
import functools

import jax
import jax.numpy as jnp
from jax import lax
from jax.experimental import pallas as pl
from jax.experimental.pallas import tpu as pltpu
from jax.experimental.pallas import tpu_sc as plsc

N = 10000
E = 320000
D = 128
OUT = 16

NC = 2
NS = 16
NW = NC * NS
EPW = E // NW
CHUNK = 128
NCHUNK = (EPW + CHUNK - 1) // CHUNK
NCHUNK = NCHUNK + (NCHUNK % 2)
EPW_PAD = NCHUNK * CHUNK
NPAD = 10240
RPT = NPAD // NS
ZCH = 128


def _sc_segment_sum(h, src2, dst2, zeros):
    mesh = plsc.VectorSubcoreMesh(core_axis_name="c", subcore_axis_name="s")

    @functools.partial(
        pl.kernel,
        out_type=jax.ShapeDtypeStruct((NC, NPAD, D), jnp.float32),
        mesh=mesh,
        scratch_types=[
            pltpu.VMEM((NCHUNK, CHUNK), jnp.int32),
            pltpu.VMEM((NCHUNK, CHUNK), jnp.int32),
            pltpu.VMEM((CHUNK, D), jnp.float32),
            pltpu.VMEM_SHARED((NPAD, D), jnp.float32),
            pltpu.SemaphoreType.DMA,
        ],
    )
    def k(h_hbm, src_hbm, dst_hbm, z_hbm, out_hbm, src_v, dst_v, rows_v, acc, sem):
        c = lax.axis_index("c")
        s = lax.axis_index("s")
        wid = c * NS + s
        pltpu.sync_copy(src_hbm.at[wid], src_v)
        pltpu.sync_copy(dst_hbm.at[wid], dst_v)
        for kk in range(RPT // ZCH):
            pltpu.sync_copy(z_hbm, acc.at[pl.ds(s * RPT + kk * ZCH, ZCH)])
        plsc.subcore_barrier()

        def body(j, carry):
            pltpu.async_copy(h_hbm.at[src_v.at[j]], rows_v, sem).wait()
            pltpu.sync_copy(rows_v, acc.at[dst_v.at[j]], add=True)
            return carry

        lax.fori_loop(0, NCHUNK, body, 0)
        plsc.subcore_barrier()
        pltpu.sync_copy(acc.at[pl.ds(s * RPT, RPT)],
                        out_hbm.at[c].at[pl.ds(s * RPT, RPT)])

    return k(h, src2, dst2, zeros)


BLK = 2000
NBLK = N // BLK


def _tc_combine(h, p):
    def body(h_ref, p0_ref, p1_ref, o_ref, acc_ref):
        ph = pl.program_id(0)
        blk = pl.program_id(1)

        @pl.when(ph == 0)
        def _():
            @pl.when(blk == 0)
            def _():
                acc_ref[...] = jnp.zeros_like(acc_ref)
            acc_ref[...] += jnp.sum(h_ref[...], axis=0, keepdims=True)

        @pl.when(ph == 1)
        def _():
            o_ref[...] = (2.0 * h_ref[...] + p0_ref[0] + p1_ref[0]
                          + acc_ref[...] * (1.0 / N))

    return pl.pallas_call(
        body,
        grid=(2, NBLK),
        in_specs=[
            pl.BlockSpec((BLK, D), lambda ph, b: (b, 0)),
            pl.BlockSpec((1, BLK, D), lambda ph, b: (0, b, 0)),
            pl.BlockSpec((1, BLK, D), lambda ph, b: (1, b, 0)),
        ],
        out_specs=pl.BlockSpec((BLK, D), lambda ph, b: (b, 0)),
        out_shape=jax.ShapeDtypeStruct((N, D), jnp.float32),
        scratch_shapes=[pltpu.VMEM((1, D), jnp.float32)],
    )(h, p, p)


def _tc_combine_predict(h, p, W, b):
    def body(h_ref, p0_ref, p1_ref, w_ref, b_ref, o_ref, acc_ref):
        ph = pl.program_id(0)
        blk = pl.program_id(1)

        @pl.when(ph == 0)
        def _():
            @pl.when(blk == 0)
            def _():
                acc_ref[...] = jnp.zeros_like(acc_ref)
            acc_ref[...] += jnp.sum(h_ref[...], axis=0, keepdims=True)

        @pl.when(ph == 1)
        def _():
            h2 = (2.0 * h_ref[...] + p0_ref[0] + p1_ref[0]
                  + acc_ref[...] * (1.0 / N))
            logits = jnp.dot(h2, w_ref[...],
                             preferred_element_type=jnp.float32) + b_ref[...]
            o_ref[...] = jax.nn.sigmoid(logits)

    return pl.pallas_call(
        body,
        grid=(2, NBLK),
        in_specs=[
            pl.BlockSpec((BLK, D), lambda ph, b_: (b_, 0)),
            pl.BlockSpec((1, BLK, D), lambda ph, b_: (0, b_, 0)),
            pl.BlockSpec((1, BLK, D), lambda ph, b_: (1, b_, 0)),
            pl.BlockSpec((D, OUT), lambda ph, b_: (0, 0)),
            pl.BlockSpec((1, OUT), lambda ph, b_: (0, 0)),
        ],
        out_specs=pl.BlockSpec((BLK, OUT), lambda ph, b_: (b_, 0)),
        out_shape=jax.ShapeDtypeStruct((N, OUT), jnp.float32),
        scratch_shapes=[pltpu.VMEM((1, D), jnp.float32)],
    )(h, p, p, W, b.reshape(1, OUT))


def kernel(x, edge_index, W_pred, b_pred):
    dst = edge_index[0]
    src = edge_index[1]
    src2 = jnp.pad(src.reshape(NW, EPW), ((0, 0), (0, EPW_PAD - EPW)))
    dst2 = jnp.pad(dst.reshape(NW, EPW), ((0, 0), (0, EPW_PAD - EPW)),
                   constant_values=N)
    src2 = src2.reshape(NW, NCHUNK, CHUNK)
    dst2 = dst2.reshape(NW, NCHUNK, CHUNK)
    zeros = jnp.zeros((ZCH, D), jnp.float32)

    p1 = _sc_segment_sum(x, src2, dst2, zeros)
    h1 = _tc_combine(x, p1)
    p2 = _sc_segment_sum(h1, src2, dst2, zeros)
    return _tc_combine_predict(h1, p2, W_pred, b_pred)
